# baseline (speedup 1.0000x reference)
import functools

import jax
import jax.numpy as jnp
from jax import lax
from jax.experimental import pallas as pl
from jax.experimental.pallas import tpu as pltpu
from jax.experimental.pallas import tpu_sc as plsc

_NC = 2
_NS = 16
_NW = _NC * _NS
_B = 4096
_S = 200
_V = 1000000
_E = 64
_C = 32
_ROWS_PER_W = _B // _NW
_IDX_PER_W = _ROWS_PER_W * _S
_CHUNK = 128
_NCHUNK = _IDX_PER_W // _CHUNK
_RING = 4
_L = 16
_DBITS = 11
_TBLK = 2048


def _tc_repack(tableT):
    grid = (_V + _TBLK - 1) // _TBLK

    def body(t_ref, i_ref, o_ref):
        x = t_ref[...]
        y = jax.lax.dot_general(
            x, i_ref[...], (((0,), (0,)), ((), ())),
            preferred_element_type=jnp.float32)
        o_ref[...] = jnp.concatenate([y, y], axis=1)

    return pl.pallas_call(
        body,
        grid=(grid,),
        in_specs=[pl.BlockSpec((_E, _TBLK), lambda i: (0, i),
                               pipeline_mode=pl.Buffered(buffer_count=2)),
                  pl.BlockSpec((_E, _E), lambda i: (0, 0))],
        out_specs=pl.BlockSpec((_TBLK, 2 * _E), lambda i: (i, 0),
                               pipeline_mode=pl.Buffered(buffer_count=2)),
        out_shape=jax.ShapeDtypeStruct((_V, 2 * _E), jnp.float32),
    )(tableT, jnp.eye(_E, dtype=jnp.float32))


def _sc_pool_sum(packed1, table2):
    mesh = plsc.VectorSubcoreMesh(core_axis_name="c", subcore_axis_name="s")

    @functools.partial(
        pl.kernel,
        out_type=jax.ShapeDtypeStruct((_NW * _ROWS_PER_W * _E,), jnp.float32),
        mesh=mesh,
        scratch_types=[
            pltpu.VMEM((_IDX_PER_W,), jnp.int32),
            pltpu.VMEM((_RING, _CHUNK), jnp.int32),
            pltpu.VMEM((_RING, _CHUNK), jnp.int32),
            pltpu.VMEM((_RING, _CHUNK, 128), jnp.float32),
            pltpu.VMEM_SHARED((_NS * _ROWS_PER_W, 128), jnp.float32),
            pltpu.VMEM((_ROWS_PER_W // 2 * _E,), jnp.float32),
        ] + [pltpu.SemaphoreType.DMA] * _RING,
    )
    def k(pk_hbm, tbl_hbm, out_hbm, pk_v, gbuf, dbuf, ring_v, acc_sh, out_s,
          *sems):
        cid = lax.axis_index("c")
        sid = lax.axis_index("s")
        wid = cid * _NS + sid
        base = sid * _ROWS_PER_W

        pltpu.sync_copy(pk_hbm.at[pl.ds(wid * _IDX_PER_W, _IDX_PER_W)], pk_v)

        zv = jnp.zeros((_L,), jnp.float32)
        @pl.loop(0, _CHUNK)
        def _(r):
            for g in range(128 // _L):
                ring_v[0, r, pl.ds(g * _L, _L)] = zv
        pltpu.sync_copy(ring_v.at[0], acc_sh.at[pl.ds(base, _CHUNK)])

        def unpack(j, b):
            for g in range(_CHUNK // _L):
                v = pk_v[pl.ds(j * _CHUNK + g * _L, _L)]
                gbuf[b, pl.ds(g * _L, _L)] = v >> _DBITS
                dbuf[b, pl.ds(g * _L, _L)] = v & ((1 << _DBITS) - 1)

        def fire(b):
            pltpu.async_copy(tbl_hbm.at[gbuf.at[b]], ring_v.at[b], sems[b])

        for b in range(_RING):
            unpack(b, b)
            fire(b)

        @pl.loop(0, _NCHUNK - _RING, step=_RING)
        def _(j0):
            for b in range(_RING):
                j = j0 + b
                pltpu.make_async_copy(
                    tbl_hbm.at[gbuf.at[b]], ring_v.at[b], sems[b]).wait()
                pltpu.sync_copy(ring_v.at[b], acc_sh.at[dbuf.at[b]], add=True)
                unpack(j + _RING, b)
                fire(b)

        for b in range(_RING):
            pltpu.make_async_copy(
                tbl_hbm.at[gbuf.at[b]], ring_v.at[b], sems[b]).wait()
            pltpu.sync_copy(ring_v.at[b], acc_sh.at[dbuf.at[b]], add=True)

        pltpu.sync_copy(acc_sh.at[pl.ds(base, _CHUNK)], ring_v.at[0])
        half = _ROWS_PER_W // 2
        for h in range(2):
            @pl.loop(0, half)
            def _(r):
                for g in range(_E // _L):
                    out_s[pl.ds(r * _E + g * _L, _L)] = (
                        ring_v[0, h * half + r, pl.ds(g * _L, _L)])
            pltpu.sync_copy(
                out_s,
                out_hbm.at[pl.ds((wid * _ROWS_PER_W + h * half) * _E,
                                 half * _E)])

    return k(packed1, table2)


def _tc_head(pooled_sum, W, b2):
    blk = 512

    def body(p_ref, w_ref, b_ref, o_ref):
        x = p_ref[...] * (1.0 / _S)
        logits = jnp.dot(x, w_ref[...], preferred_element_type=jnp.float32)
        logits = logits + b_ref[...]
        m = jnp.max(logits, axis=-1, keepdims=True)
        e = jnp.exp(logits - m)
        o_ref[...] = e / jnp.sum(e, axis=-1, keepdims=True)

    return pl.pallas_call(
        body,
        grid=(_B // blk,),
        in_specs=[
            pl.BlockSpec((blk, _E), lambda i: (i, 0)),
            pl.BlockSpec((_E, _C), lambda i: (0, 0)),
            pl.BlockSpec((1, _C), lambda i: (0, 0)),
        ],
        out_specs=pl.BlockSpec((blk, _C), lambda i: (i, 0)),
        out_shape=jax.ShapeDtypeStruct((_B, _C), jnp.float32),
    )(pooled_sum, W, b2)


def kernel(indices, table, W, b):
    idx = indices.astype(jnp.int32)
    table2 = _tc_repack(table.T)
    rows = jnp.arange(_B, dtype=jnp.int32)
    accrow = (rows % _ROWS_PER_W) + ((rows // _ROWS_PER_W) % _NS) * _ROWS_PER_W
    packed = (idx << _DBITS) | accrow[:, None]
    packed1 = packed.reshape(_NW * _IDX_PER_W)
    pooled_sum = _sc_pool_sum(packed1, table2).reshape(_B, _E)
    return _tc_head(pooled_sum, W, b.reshape(1, _C))


# speedup vs baseline: 1.3320x; 1.3320x over previous
import functools

import jax
import jax.numpy as jnp
from jax import lax
from jax.experimental import pallas as pl
from jax.experimental.pallas import tpu as pltpu
from jax.experimental.pallas import tpu_sc as plsc

_NC = 2
_NS = 16
_NW = _NC * _NS
_B = 4096
_S = 200
_V = 1000000
_E = 64
_C = 32
_ROWS_PER_W = _B // _NW
_IDX_PER_W = _ROWS_PER_W * _S
_CHUNK = 64
_NCHUNK = _IDX_PER_W // _CHUNK
_RING = 8
_L = 16
_DBITS = 11
_TBLK = 8192


def _tc_repack(tableT):
    grid = (_V + _TBLK - 1) // _TBLK

    def body(t_ref, i_ref, o_ref):
        x = t_ref[...]
        y = jax.lax.dot_general(
            x, i_ref[...], (((0,), (0,)), ((), ())),
            preferred_element_type=jnp.float32)
        o_ref[...] = jnp.concatenate([y, y], axis=1)

    return pl.pallas_call(
        body,
        grid=(grid,),
        in_specs=[pl.BlockSpec((_E, _TBLK), lambda i: (0, i),
                               pipeline_mode=pl.Buffered(buffer_count=2)),
                  pl.BlockSpec((_E, _E), lambda i: (0, 0))],
        out_specs=pl.BlockSpec((_TBLK, 2 * _E), lambda i: (i, 0),
                               pipeline_mode=pl.Buffered(buffer_count=2)),
        out_shape=jax.ShapeDtypeStruct((_V, 2 * _E), jnp.float32),
    )(tableT, jnp.eye(_E, dtype=jnp.float32))


def _sc_pool_sum(packed1, table2):
    mesh = plsc.VectorSubcoreMesh(core_axis_name="c", subcore_axis_name="s")
    hr = _RING // 2

    @functools.partial(
        pl.kernel,
        out_type=jax.ShapeDtypeStruct((_B * _E,), jnp.float32),
        mesh=mesh,
        scratch_types=[
            pltpu.VMEM((_IDX_PER_W,), jnp.int32),
            pltpu.VMEM((_RING, _CHUNK), jnp.int32),
            pltpu.VMEM((_RING, _CHUNK), jnp.int32),
            pltpu.VMEM((_RING, _CHUNK, 128), jnp.float32),
            pltpu.VMEM_SHARED((_NS * _ROWS_PER_W, 128), jnp.float32),
            pltpu.VMEM((_ROWS_PER_W // 2 * _E,), jnp.float32),
        ] + [pltpu.SemaphoreType.DMA] * (2 * _RING),
    )
    def k(pk_hbm, tbl_hbm, out_hbm, pk_v, gbuf, dbuf, ring_v, acc_sh, out_s,
          *sems):
        gsem = sems[:_RING]
        ssem = sems[_RING:]
        cid = lax.axis_index("c")
        sid = lax.axis_index("s")
        wid = cid * _NS + sid
        base = sid * _ROWS_PER_W

        pltpu.sync_copy(pk_hbm.at[pl.ds(wid * _IDX_PER_W, _IDX_PER_W)], pk_v)

        zv = jnp.zeros((_L,), jnp.float32)
        @pl.loop(0, _CHUNK)
        def _(r):
            for g in range(128 // _L):
                ring_v[0, r, pl.ds(g * _L, _L)] = zv
                ring_v[1, r, pl.ds(g * _L, _L)] = zv
        pltpu.sync_copy(ring_v.at[0], acc_sh.at[pl.ds(base, _CHUNK)])
        pltpu.sync_copy(ring_v.at[1], acc_sh.at[pl.ds(base + _CHUNK, _CHUNK)])

        def unpack(j, b):
            for g in range(_CHUNK // _L):
                v = pk_v[pl.ds(j * _CHUNK + g * _L, _L)]
                gbuf[b, pl.ds(g * _L, _L)] = v >> _DBITS
                dbuf[b, pl.ds(g * _L, _L)] = v & ((1 << _DBITS) - 1)

        def fire_g(b):
            pltpu.async_copy(tbl_hbm.at[gbuf.at[b]], ring_v.at[b], gsem[b])

        def wait_g(b):
            pltpu.make_async_copy(
                tbl_hbm.at[gbuf.at[b]], ring_v.at[b], gsem[b]).wait()

        def fire_s(b):
            pltpu.async_copy(
                ring_v.at[b], acc_sh.at[dbuf.at[b]], ssem[b], add=True)

        def wait_s(b):
            pltpu.make_async_copy(
                ring_v.at[b], acc_sh.at[dbuf.at[b]], ssem[b]).wait()

        for b in range(_RING):
            unpack(b, b)
            fire_g(b)
        for b in range(hr):
            wait_g(b)
            fire_s(b)

        @pl.loop(_RING, _NCHUNK, step=_RING)
        def _(j0):
            for b in range(_RING):
                j = j0 + b
                wait_g((b + hr) % _RING)
                fire_s((b + hr) % _RING)
                wait_s(b)
                unpack(j, b)
                fire_g(b)

        for b in range(hr):
            wait_g((b + hr) % _RING)
            fire_s((b + hr) % _RING)
        for b in range(_RING):
            wait_s(b)

        pltpu.sync_copy(acc_sh.at[pl.ds(base, _CHUNK)], ring_v.at[0])
        pltpu.sync_copy(acc_sh.at[pl.ds(base + _CHUNK, _CHUNK)], ring_v.at[1])
        half = _ROWS_PER_W // 2
        for h in range(2):
            @pl.loop(0, half)
            def _(r):
                for g in range(_E // _L):
                    out_s[pl.ds(r * _E + g * _L, _L)] = (
                        ring_v[h, r, pl.ds(g * _L, _L)])
            pltpu.sync_copy(
                out_s,
                out_hbm.at[pl.ds((wid * _ROWS_PER_W + h * half) * _E,
                                 half * _E)])

    return k(packed1, table2)


def _tc_head(pooled_sum, W, b2):
    blk = 512

    def body(p_ref, w_ref, b_ref, o_ref):
        x = p_ref[...] * (1.0 / _S)
        logits = jnp.dot(x, w_ref[...], preferred_element_type=jnp.float32)
        logits = logits + b_ref[...]
        m = jnp.max(logits, axis=-1, keepdims=True)
        e = jnp.exp(logits - m)
        o_ref[...] = e / jnp.sum(e, axis=-1, keepdims=True)

    return pl.pallas_call(
        body,
        grid=(_B // blk,),
        in_specs=[
            pl.BlockSpec((blk, _E), lambda i: (i, 0)),
            pl.BlockSpec((_E, _C), lambda i: (0, 0)),
            pl.BlockSpec((1, _C), lambda i: (0, 0)),
        ],
        out_specs=pl.BlockSpec((blk, _C), lambda i: (i, 0)),
        out_shape=jax.ShapeDtypeStruct((_B, _C), jnp.float32),
    )(pooled_sum, W, b2)


def kernel(indices, table, W, b):
    idx = indices.astype(jnp.int32)
    table2 = _tc_repack(table.T)
    rows = jnp.arange(_B, dtype=jnp.int32)
    accrow = (rows % _ROWS_PER_W) + ((rows // _ROWS_PER_W) % _NS) * _ROWS_PER_W
    packed = (idx << _DBITS) | accrow[:, None]
    packed1 = packed.reshape(_NW * _IDX_PER_W)
    pooled_sum = _sc_pool_sum(packed1, table2).reshape(_B, _E)
    return _tc_head(pooled_sum, W, b.reshape(1, _C))


# speedup vs baseline: 1.4103x; 1.0588x over previous
import functools

import jax
import jax.numpy as jnp
from jax import lax
from jax.experimental import pallas as pl
from jax.experimental.pallas import tpu as pltpu
from jax.experimental.pallas import tpu_sc as plsc

_NC = 2
_NS = 16
_NW = _NC * _NS
_B = 4096
_S = 200
_V = 1000000
_E = 64
_C = 32
_ROWS_PER_W = _B // _NW
_IDX_PER_W = _ROWS_PER_W * _S
_CHUNK = 64
_NCHUNK = _IDX_PER_W // _CHUNK
_RING = 8
_L = 16
_DBITS = 11
_TBLK = 16384


def _tc_repack(tableT):
    grid = (_V + _TBLK - 1) // _TBLK

    def body(t_ref, i_ref, o_ref):
        x = t_ref[...]
        y = jax.lax.dot_general(
            x, i_ref[...], (((0,), (0,)), ((), ())),
            preferred_element_type=jnp.float32)
        o_ref[...] = jnp.concatenate([y, y], axis=1)

    return pl.pallas_call(
        body,
        grid=(grid,),
        in_specs=[pl.BlockSpec((_E, _TBLK), lambda i: (0, i),
                               pipeline_mode=pl.Buffered(buffer_count=2)),
                  pl.BlockSpec((_E, _E), lambda i: (0, 0))],
        out_specs=pl.BlockSpec((_TBLK, 2 * _E), lambda i: (i, 0),
                               pipeline_mode=pl.Buffered(buffer_count=2)),
        out_shape=jax.ShapeDtypeStruct((_V, 2 * _E), jnp.float32),
    )(tableT, jnp.eye(_E, dtype=jnp.float32))


def _sc_pool_sum(packed1, table2):
    mesh = plsc.VectorSubcoreMesh(core_axis_name="c", subcore_axis_name="s")
    hr = _RING // 2

    @functools.partial(
        pl.kernel,
        out_type=jax.ShapeDtypeStruct((_B * _E,), jnp.float32),
        mesh=mesh,
        scratch_types=[
            pltpu.VMEM((_IDX_PER_W,), jnp.int32),
            pltpu.VMEM((_RING, _CHUNK), jnp.int32),
            pltpu.VMEM((_RING, _CHUNK), jnp.int32),
            pltpu.VMEM((_RING, _CHUNK, 128), jnp.float32),
            pltpu.VMEM_SHARED((_NS * _ROWS_PER_W, 128), jnp.float32),
            pltpu.VMEM((_ROWS_PER_W // 2 * _E,), jnp.float32),
        ] + [pltpu.SemaphoreType.DMA] * (2 * _RING),
    )
    def k(pk_hbm, tbl_hbm, out_hbm, pk_v, gbuf, dbuf, ring_v, acc_sh, out_s,
          *sems):
        gsem = sems[:_RING]
        ssem = sems[_RING:]
        cid = lax.axis_index("c")
        sid = lax.axis_index("s")
        wid = cid * _NS + sid
        base = sid * _ROWS_PER_W

        pltpu.sync_copy(pk_hbm.at[pl.ds(wid * _IDX_PER_W, _IDX_PER_W)], pk_v)

        zv = jnp.zeros((_L,), jnp.float32)
        @pl.loop(0, _CHUNK)
        def _(r):
            for g in range(128 // _L):
                ring_v[0, r, pl.ds(g * _L, _L)] = zv
                ring_v[1, r, pl.ds(g * _L, _L)] = zv
        pltpu.sync_copy(ring_v.at[0], acc_sh.at[pl.ds(base, _CHUNK)])
        pltpu.sync_copy(ring_v.at[1], acc_sh.at[pl.ds(base + _CHUNK, _CHUNK)])

        def unpack(j, b):
            for g in range(_CHUNK // _L):
                v = pk_v[pl.ds(j * _CHUNK + g * _L, _L)]
                gbuf[b, pl.ds(g * _L, _L)] = v >> _DBITS
                dbuf[b, pl.ds(g * _L, _L)] = v & ((1 << _DBITS) - 1)

        def fire_g(b):
            pltpu.async_copy(tbl_hbm.at[gbuf.at[b]], ring_v.at[b], gsem[b])

        def wait_g(b):
            pltpu.make_async_copy(
                tbl_hbm.at[gbuf.at[b]], ring_v.at[b], gsem[b]).wait()

        def fire_s(b):
            pltpu.async_copy(
                ring_v.at[b], acc_sh.at[dbuf.at[b]], ssem[b], add=True)

        def wait_s(b):
            pltpu.make_async_copy(
                ring_v.at[b], acc_sh.at[dbuf.at[b]], ssem[b]).wait()

        for b in range(_RING):
            unpack(b, b)
            fire_g(b)
        for b in range(hr):
            wait_g(b)
            fire_s(b)

        @pl.loop(_RING, _NCHUNK, step=_RING)
        def _(j0):
            for b in range(_RING):
                j = j0 + b
                wait_g((b + hr) % _RING)
                fire_s((b + hr) % _RING)
                wait_s(b)
                unpack(j, b)
                fire_g(b)

        for b in range(hr):
            wait_g((b + hr) % _RING)
            fire_s((b + hr) % _RING)
        for b in range(_RING):
            wait_s(b)

        pltpu.sync_copy(acc_sh.at[pl.ds(base, _CHUNK)], ring_v.at[0])
        pltpu.sync_copy(acc_sh.at[pl.ds(base + _CHUNK, _CHUNK)], ring_v.at[1])
        half = _ROWS_PER_W // 2
        for h in range(2):
            @pl.loop(0, half)
            def _(r):
                for g in range(_E // _L):
                    out_s[pl.ds(r * _E + g * _L, _L)] = (
                        ring_v[h, r, pl.ds(g * _L, _L)])
            pltpu.sync_copy(
                out_s,
                out_hbm.at[pl.ds((wid * _ROWS_PER_W + h * half) * _E,
                                 half * _E)])

    return k(packed1, table2)


def _tc_head(pooled_sum, W, b2):
    blk = 512

    def body(p_ref, w_ref, b_ref, o_ref):
        x = p_ref[...] * (1.0 / _S)
        logits = jnp.dot(x, w_ref[...], preferred_element_type=jnp.float32)
        logits = logits + b_ref[...]
        m = jnp.max(logits, axis=-1, keepdims=True)
        e = jnp.exp(logits - m)
        o_ref[...] = e / jnp.sum(e, axis=-1, keepdims=True)

    return pl.pallas_call(
        body,
        grid=(_B // blk,),
        in_specs=[
            pl.BlockSpec((blk, _E), lambda i: (i, 0)),
            pl.BlockSpec((_E, _C), lambda i: (0, 0)),
            pl.BlockSpec((1, _C), lambda i: (0, 0)),
        ],
        out_specs=pl.BlockSpec((blk, _C), lambda i: (i, 0)),
        out_shape=jax.ShapeDtypeStruct((_B, _C), jnp.float32),
    )(pooled_sum, W, b2)


def kernel(indices, table, W, b):
    idx = indices.astype(jnp.int32)
    table2 = _tc_repack(table.T)
    rows = jnp.arange(_B, dtype=jnp.int32)
    accrow = (rows % _ROWS_PER_W) + ((rows // _ROWS_PER_W) % _NS) * _ROWS_PER_W
    packed = (idx << _DBITS) | accrow[:, None]
    packed1 = packed.reshape(_NW * _IDX_PER_W)
    pooled_sum = _sc_pool_sum(packed1, table2).reshape(_B, _E)
    return _tc_head(pooled_sum, W, b.reshape(1, _C))


# speedup vs baseline: 1.5309x; 1.0856x over previous
import functools

import jax
import jax.numpy as jnp
from jax import lax
from jax.experimental import pallas as pl
from jax.experimental.pallas import tpu as pltpu
from jax.experimental.pallas import tpu_sc as plsc

_NC = 2
_NS = 16
_NW = _NC * _NS
_B = 4096
_S = 200
_V = 1000000
_E = 64
_C = 32
_ROWS_PER_W = _B // _NW
_IDX_PER_W = _ROWS_PER_W * _S
_CHUNK = 64
_NCHUNK = _IDX_PER_W // _CHUNK
_RING = 8
_L = 16
_DBITS = 11
_TBLK = 16384


def _tc_repack(tableT):
    grid = (_V + _TBLK - 1) // _TBLK

    def body(t_ref, i_ref, o_ref):
        x = t_ref[...]
        y = jax.lax.dot_general(
            x, i_ref[...], (((0,), (0,)), ((), ())),
            preferred_element_type=jnp.float32)
        o_ref[:, 0:_E] = y

    return pl.pallas_call(
        body,
        grid=(grid,),
        in_specs=[pl.BlockSpec((_E, _TBLK), lambda i: (0, i),
                               pipeline_mode=pl.Buffered(buffer_count=2)),
                  pl.BlockSpec((_E, _E), lambda i: (0, 0))],
        out_specs=pl.BlockSpec((_TBLK, 2 * _E), lambda i: (i, 0),
                               pipeline_mode=pl.Buffered(buffer_count=2)),
        out_shape=jax.ShapeDtypeStruct((_V, 2 * _E), jnp.float32),
    )(tableT, jnp.eye(_E, dtype=jnp.float32))


def _sc_pool_sum(packed1, table2):
    mesh = plsc.VectorSubcoreMesh(core_axis_name="c", subcore_axis_name="s")
    hr = _RING // 2

    @functools.partial(
        pl.kernel,
        out_type=jax.ShapeDtypeStruct((_B * _E,), jnp.float32),
        mesh=mesh,
        scratch_types=[
            pltpu.VMEM((_IDX_PER_W,), jnp.int32),
            pltpu.VMEM((_RING, _CHUNK), jnp.int32),
            pltpu.VMEM((_RING, _CHUNK), jnp.int32),
            pltpu.VMEM((_RING, _CHUNK, 128), jnp.float32),
            pltpu.VMEM_SHARED((_NS * _ROWS_PER_W, 128), jnp.float32),
            pltpu.VMEM((_ROWS_PER_W // 2 * _E,), jnp.float32),
        ] + [pltpu.SemaphoreType.DMA] * (2 * _RING),
    )
    def k(pk_hbm, tbl_hbm, out_hbm, pk_v, gbuf, dbuf, ring_v, acc_sh, out_s,
          *sems):
        gsem = sems[:_RING]
        ssem = sems[_RING:]
        cid = lax.axis_index("c")
        sid = lax.axis_index("s")
        wid = cid * _NS + sid
        base = sid * _ROWS_PER_W

        pltpu.sync_copy(pk_hbm.at[pl.ds(wid * _IDX_PER_W, _IDX_PER_W)], pk_v)

        zv = jnp.zeros((_L,), jnp.float32)
        @pl.loop(0, _CHUNK)
        def _(r):
            for g in range(128 // _L):
                ring_v[0, r, pl.ds(g * _L, _L)] = zv
                ring_v[1, r, pl.ds(g * _L, _L)] = zv
        pltpu.sync_copy(ring_v.at[0], acc_sh.at[pl.ds(base, _CHUNK)])
        pltpu.sync_copy(ring_v.at[1], acc_sh.at[pl.ds(base + _CHUNK, _CHUNK)])

        def unpack(j, b):
            for g in range(_CHUNK // _L):
                v = pk_v[pl.ds(j * _CHUNK + g * _L, _L)]
                gbuf[b, pl.ds(g * _L, _L)] = v >> _DBITS
                dbuf[b, pl.ds(g * _L, _L)] = v & ((1 << _DBITS) - 1)

        def fire_g(b):
            pltpu.async_copy(tbl_hbm.at[gbuf.at[b]], ring_v.at[b], gsem[b])

        def wait_g(b):
            pltpu.make_async_copy(
                tbl_hbm.at[gbuf.at[b]], ring_v.at[b], gsem[b]).wait()

        def fire_s(b):
            pltpu.async_copy(
                ring_v.at[b], acc_sh.at[dbuf.at[b]], ssem[b], add=True)

        def wait_s(b):
            pltpu.make_async_copy(
                ring_v.at[b], acc_sh.at[dbuf.at[b]], ssem[b]).wait()

        for b in range(_RING):
            unpack(b, b)
            fire_g(b)
        for b in range(hr):
            wait_g(b)
            fire_s(b)

        @pl.loop(_RING, _NCHUNK, step=_RING)
        def _(j0):
            for b in range(_RING):
                j = j0 + b
                wait_g((b + hr) % _RING)
                fire_s((b + hr) % _RING)
                wait_s(b)
                unpack(j, b)
                fire_g(b)

        for b in range(hr):
            wait_g((b + hr) % _RING)
            fire_s((b + hr) % _RING)
        for b in range(_RING):
            wait_s(b)

        pltpu.sync_copy(acc_sh.at[pl.ds(base, _CHUNK)], ring_v.at[0])
        pltpu.sync_copy(acc_sh.at[pl.ds(base + _CHUNK, _CHUNK)], ring_v.at[1])
        half = _ROWS_PER_W // 2
        for h in range(2):
            @pl.loop(0, half)
            def _(r):
                for g in range(_E // _L):
                    out_s[pl.ds(r * _E + g * _L, _L)] = (
                        ring_v[h, r, pl.ds(g * _L, _L)])
            pltpu.sync_copy(
                out_s,
                out_hbm.at[pl.ds((wid * _ROWS_PER_W + h * half) * _E,
                                 half * _E)])

    return k(packed1, table2)


def _tc_head(pooled_sum, W, b2):
    blk = 512

    def body(p_ref, w_ref, b_ref, o_ref):
        x = p_ref[...] * (1.0 / _S)
        logits = jnp.dot(x, w_ref[...], preferred_element_type=jnp.float32)
        logits = logits + b_ref[...]
        m = jnp.max(logits, axis=-1, keepdims=True)
        e = jnp.exp(logits - m)
        o_ref[...] = e / jnp.sum(e, axis=-1, keepdims=True)

    return pl.pallas_call(
        body,
        grid=(_B // blk,),
        in_specs=[
            pl.BlockSpec((blk, _E), lambda i: (i, 0)),
            pl.BlockSpec((_E, _C), lambda i: (0, 0)),
            pl.BlockSpec((1, _C), lambda i: (0, 0)),
        ],
        out_specs=pl.BlockSpec((blk, _C), lambda i: (i, 0)),
        out_shape=jax.ShapeDtypeStruct((_B, _C), jnp.float32),
    )(pooled_sum, W, b2)


def kernel(indices, table, W, b):
    idx = indices.astype(jnp.int32)
    table2 = _tc_repack(table.T)
    rows = jnp.arange(_B, dtype=jnp.int32)
    accrow = (rows % _ROWS_PER_W) + ((rows // _ROWS_PER_W) % _NS) * _ROWS_PER_W
    packed = (idx << _DBITS) | accrow[:, None]
    packed1 = packed.reshape(_NW * _IDX_PER_W)
    pooled_sum = _sc_pool_sum(packed1, table2).reshape(_B, _E)
    return _tc_head(pooled_sum, W, b.reshape(1, _C))


# speedup vs baseline: 1.5418x; 1.0071x over previous
import functools

import jax
import jax.numpy as jnp
from jax import lax
from jax.experimental import pallas as pl
from jax.experimental.pallas import tpu as pltpu
from jax.experimental.pallas import tpu_sc as plsc

_NC = 2
_NS = 16
_NW = _NC * _NS
_B = 4096
_S = 200
_V = 1000000
_E = 64
_C = 32
_ROWS_PER_W = _B // _NW
_IDX_PER_W = _ROWS_PER_W * _S
_CHUNK = 64
_NCHUNK = _IDX_PER_W // _CHUNK
_RING = 8
_L = 16
_DBITS = 11
_TBLK = 32768


def _tc_repack(tableT):
    grid = (_V + _TBLK - 1) // _TBLK

    def body(t_ref, i_ref, o_ref):
        x = t_ref[...]
        y = jax.lax.dot_general(
            x, i_ref[...], (((0,), (0,)), ((), ())),
            preferred_element_type=jnp.float32)
        o_ref[:, 0:_E] = y

    return pl.pallas_call(
        body,
        grid=(grid,),
        in_specs=[pl.BlockSpec((_E, _TBLK), lambda i: (0, i),
                               pipeline_mode=pl.Buffered(buffer_count=2)),
                  pl.BlockSpec((_E, _E), lambda i: (0, 0))],
        out_specs=pl.BlockSpec((_TBLK, 2 * _E), lambda i: (i, 0),
                               pipeline_mode=pl.Buffered(buffer_count=2)),
        out_shape=jax.ShapeDtypeStruct((_V, 2 * _E), jnp.float32),
    )(tableT, jnp.eye(_E, dtype=jnp.float32))


def _sc_pool_sum(packed1, table2):
    mesh = plsc.VectorSubcoreMesh(core_axis_name="c", subcore_axis_name="s")
    hr = _RING // 2

    @functools.partial(
        pl.kernel,
        out_type=jax.ShapeDtypeStruct((_B * _E,), jnp.float32),
        mesh=mesh,
        scratch_types=[
            pltpu.VMEM((_IDX_PER_W,), jnp.int32),
            pltpu.VMEM((_RING, _CHUNK), jnp.int32),
            pltpu.VMEM((_RING, _CHUNK), jnp.int32),
            pltpu.VMEM((_RING, _CHUNK, 128), jnp.float32),
            pltpu.VMEM_SHARED((_NS * _ROWS_PER_W, 128), jnp.float32),
            pltpu.VMEM((_ROWS_PER_W // 2 * _E,), jnp.float32),
        ] + [pltpu.SemaphoreType.DMA] * (2 * _RING),
    )
    def k(pk_hbm, tbl_hbm, out_hbm, pk_v, gbuf, dbuf, ring_v, acc_sh, out_s,
          *sems):
        gsem = sems[:_RING]
        ssem = sems[_RING:]
        cid = lax.axis_index("c")
        sid = lax.axis_index("s")
        wid = cid * _NS + sid
        base = sid * _ROWS_PER_W

        pltpu.sync_copy(pk_hbm.at[pl.ds(wid * _IDX_PER_W, _IDX_PER_W)], pk_v)

        zv = jnp.zeros((_L,), jnp.float32)
        @pl.loop(0, _CHUNK)
        def _(r):
            for g in range(128 // _L):
                ring_v[0, r, pl.ds(g * _L, _L)] = zv
                ring_v[1, r, pl.ds(g * _L, _L)] = zv
        pltpu.sync_copy(ring_v.at[0], acc_sh.at[pl.ds(base, _CHUNK)])
        pltpu.sync_copy(ring_v.at[1], acc_sh.at[pl.ds(base + _CHUNK, _CHUNK)])

        def unpack(j, b):
            for g in range(_CHUNK // _L):
                v = pk_v[pl.ds(j * _CHUNK + g * _L, _L)]
                gbuf[b, pl.ds(g * _L, _L)] = v >> _DBITS
                dbuf[b, pl.ds(g * _L, _L)] = v & ((1 << _DBITS) - 1)

        def fire_g(b):
            pltpu.async_copy(tbl_hbm.at[gbuf.at[b]], ring_v.at[b], gsem[b])

        def wait_g(b):
            pltpu.make_async_copy(
                tbl_hbm.at[gbuf.at[b]], ring_v.at[b], gsem[b]).wait()

        def fire_s(b):
            pltpu.async_copy(
                ring_v.at[b], acc_sh.at[dbuf.at[b]], ssem[b], add=True)

        def wait_s(b):
            pltpu.make_async_copy(
                ring_v.at[b], acc_sh.at[dbuf.at[b]], ssem[b]).wait()

        for b in range(_RING):
            unpack(b, b)
            fire_g(b)
        for b in range(hr):
            wait_g(b)
            fire_s(b)

        @pl.loop(_RING, _NCHUNK, step=_RING)
        def _(j0):
            for b in range(_RING):
                j = j0 + b
                wait_g((b + hr) % _RING)
                fire_s((b + hr) % _RING)
                wait_s(b)
                unpack(j, b)
                fire_g(b)

        for b in range(hr):
            wait_g((b + hr) % _RING)
            fire_s((b + hr) % _RING)
        for b in range(_RING):
            wait_s(b)

        pltpu.sync_copy(acc_sh.at[pl.ds(base, _CHUNK)], ring_v.at[0])
        pltpu.sync_copy(acc_sh.at[pl.ds(base + _CHUNK, _CHUNK)], ring_v.at[1])
        half = _ROWS_PER_W // 2
        for h in range(2):
            @pl.loop(0, half)
            def _(r):
                for g in range(_E // _L):
                    out_s[pl.ds(r * _E + g * _L, _L)] = (
                        ring_v[h, r, pl.ds(g * _L, _L)])
            pltpu.sync_copy(
                out_s,
                out_hbm.at[pl.ds((wid * _ROWS_PER_W + h * half) * _E,
                                 half * _E)])

    return k(packed1, table2)


def _tc_head(pooled_sum, W, b2):
    blk = 512

    def body(p_ref, w_ref, b_ref, o_ref):
        x = p_ref[...] * (1.0 / _S)
        logits = jnp.dot(x, w_ref[...], preferred_element_type=jnp.float32)
        logits = logits + b_ref[...]
        m = jnp.max(logits, axis=-1, keepdims=True)
        e = jnp.exp(logits - m)
        o_ref[...] = e / jnp.sum(e, axis=-1, keepdims=True)

    return pl.pallas_call(
        body,
        grid=(_B // blk,),
        in_specs=[
            pl.BlockSpec((blk, _E), lambda i: (i, 0)),
            pl.BlockSpec((_E, _C), lambda i: (0, 0)),
            pl.BlockSpec((1, _C), lambda i: (0, 0)),
        ],
        out_specs=pl.BlockSpec((blk, _C), lambda i: (i, 0)),
        out_shape=jax.ShapeDtypeStruct((_B, _C), jnp.float32),
    )(pooled_sum, W, b2)


def kernel(indices, table, W, b):
    idx = indices.astype(jnp.int32)
    table2 = _tc_repack(table.T)
    rows = jnp.arange(_B, dtype=jnp.int32)
    accrow = (rows % _ROWS_PER_W) + ((rows // _ROWS_PER_W) % _NS) * _ROWS_PER_W
    packed = (idx << _DBITS) | accrow[:, None]
    packed1 = packed.reshape(_NW * _IDX_PER_W)
    pooled_sum = _sc_pool_sum(packed1, table2).reshape(_B, _E)
    return _tc_head(pooled_sum, W, b.reshape(1, _C))
